# baseline (speedup 1.0000x reference)
import functools

import jax
import jax.numpy as jnp
from jax import lax
from jax.experimental import pallas as pl
from jax.experimental.pallas import tpu as pltpu
from jax.experimental.pallas import tpu_sc as plsc

N, C, H, W = 1000, 81, 28, 28
D = H * W
LANES = 16
NUM_CORES = 2
NUM_SUBCORES = 16
NW = NUM_CORES * NUM_SUBCORES
NPAD = 1024
BPW = NPAD // NW


def _sc_body(x_hbm, lab_hbm, out_hbm, lab_v, idx_v, rows_v, sem):
    c = lax.axis_index("c")
    s = lax.axis_index("s")
    wid = s * NUM_CORES + c
    base = wid * BPW

    pltpu.sync_copy(lab_hbm.at[pl.ds(base, BPW)], lab_v)

    for k in range(BPW // LANES):
        lab = lab_v[pl.ds(k * LANES, LANES)]
        det = base + k * LANES + lax.iota(jnp.int32, LANES)
        idx = jnp.minimum(det * C + lab, N * C - 1)
        idx_v[pl.ds(k * LANES, LANES)] = idx

    pltpu.async_copy(x_hbm.at[idx_v], rows_v, sem).wait()

    def row_body(j, _):
        for k in range(D // LANES):
            v = rows_v[j, pl.ds(k * LANES, LANES)]
            rows_v[j, pl.ds(k * LANES, LANES)] = 1.0 / (1.0 + jnp.exp(-v))
        return 0

    lax.fori_loop(0, BPW, row_body, 0)

    pltpu.sync_copy(rows_v, out_hbm.at[pl.ds(base, BPW)])


@jax.jit
def _gather_sigmoid(x_rows, labels_padded):
    mesh = plsc.VectorSubcoreMesh(core_axis_name="c", subcore_axis_name="s")
    return pl.kernel(
        _sc_body,
        out_type=jax.ShapeDtypeStruct((NPAD, D), jnp.float32),
        mesh=mesh,
        scratch_types=[
            pltpu.VMEM((BPW,), jnp.int32),
            pltpu.VMEM((BPW,), jnp.int32),
            pltpu.VMEM((BPW, D), jnp.float32),
            pltpu.SemaphoreType.DMA,
        ],
        compiler_params=pltpu.CompilerParams(use_tc_tiling_on_sc=False),
    )(x_rows, labels_padded)


def kernel(x, labels):
    x_rows = x.reshape(N * C, D)
    lab = jnp.pad(labels.astype(jnp.int32), (0, NPAD - N))
    out = _gather_sigmoid(x_rows, lab)
    return out[:N].reshape(N, 1, H, W)


# speedup vs baseline: 2.3555x; 2.3555x over previous
import jax
import jax.numpy as jnp
from jax import lax
from jax.experimental import pallas as pl
from jax.experimental.pallas import tpu as pltpu
from jax.experimental.pallas import tpu_sc as plsc

N, C, H, W = 1000, 81, 28, 28
LANES = 16
NUM_CORES = 2
NUM_SUBCORES = 16
NW = NUM_CORES * NUM_SUBCORES
NPAD = 1024
BPW = NPAD // NW
CH = 16


def _sc_body(x_hbm, lab_hbm, out_hbm, lab_v, rows_v, sem):
    c = lax.axis_index("c")
    s = lax.axis_index("s")
    wid = s * NUM_CORES + c
    base = wid * BPW

    pltpu.sync_copy(lab_hbm.at[pl.ds(base, BPW)], lab_v)

    for chunk in range(BPW // CH):
        vv = lab_v[pl.ds(chunk * CH, LANES)]
        for l in range(CH):
            j = chunk * CH + l
            plane = jnp.minimum((base + j) * C + vv[l], N * C - 1)
            pltpu.async_copy(x_hbm.at[plane], rows_v.at[l], sem)

        def drain(j, _):
            pltpu.make_async_copy(x_hbm.at[0], rows_v.at[0], sem).wait()
            return 0

        lax.fori_loop(0, CH, drain, 0)

        def plane_body(j, _):
            for r in range(H):
                v1 = rows_v[j, r, pl.ds(0, LANES)]
                v2 = rows_v[j, r, pl.ds(W - LANES, LANES)]
                rows_v[j, r, pl.ds(0, LANES)] = 1.0 / (1.0 + jnp.exp(-v1))
                rows_v[j, r, pl.ds(W - LANES, LANES)] = (
                    1.0 / (1.0 + jnp.exp(-v2)))
            return 0

        lax.fori_loop(0, CH, plane_body, 0)

        pltpu.sync_copy(rows_v, out_hbm.at[pl.ds(base + chunk * CH, CH)])


@jax.jit
def _gather_sigmoid(x_planes, labels_padded):
    mesh = plsc.VectorSubcoreMesh(core_axis_name="c", subcore_axis_name="s")
    return pl.kernel(
        _sc_body,
        out_type=jax.ShapeDtypeStruct((NPAD, H, W), jnp.float32),
        mesh=mesh,
        scratch_types=[
            pltpu.VMEM((BPW,), jnp.int32),
            pltpu.VMEM((CH, H, W), jnp.float32),
            pltpu.SemaphoreType.DMA,
        ],
    )(x_planes, labels_padded)


def kernel(x, labels):
    x_planes = x.reshape(N * C, H, W)
    lab = jnp.pad(labels.astype(jnp.int32), (0, NPAD - N))
    out = _gather_sigmoid(x_planes, lab)
    return out[:N].reshape(N, 1, H, W)


# speedup vs baseline: 2.3852x; 1.0126x over previous
import jax
import jax.numpy as jnp
from jax import lax
from jax.experimental import pallas as pl
from jax.experimental.pallas import tpu as pltpu
from jax.experimental.pallas import tpu_sc as plsc

N, C, H, W = 1000, 81, 28, 28
LANES = 16
NUM_CORES = 2
NUM_SUBCORES = 16
NW = NUM_CORES * NUM_SUBCORES
NPAD = 1024
BPW = NPAD // NW
CH = 16


def _sc_body(x_hbm, lab_hbm, out_hbm, lab_v, rows_v, sem):
    c = lax.axis_index("c")
    s = lax.axis_index("s")
    wid = s * NUM_CORES + c
    base = wid * BPW

    pltpu.sync_copy(lab_hbm.at[pl.ds(base, BPW)], lab_v)

    for chunk in range(BPW // CH):
        vv = lab_v[pl.ds(chunk * CH, LANES)]
        for l in range(CH):
            j = chunk * CH + l
            det = jnp.minimum(base + j, N - 1)
            pltpu.async_copy(x_hbm.at[det, vv[l]], rows_v.at[l], sem)

        def drain(j, _):
            pltpu.make_async_copy(x_hbm.at[0, 0], rows_v.at[0], sem).wait()
            return 0

        lax.fori_loop(0, CH, drain, 0)

        def plane_body(j, _):
            for r in range(H):
                v1 = rows_v[j, r, pl.ds(0, LANES)]
                v2 = rows_v[j, r, pl.ds(W - LANES, LANES)]
                rows_v[j, r, pl.ds(0, LANES)] = 1.0 / (1.0 + jnp.exp(-v1))
                rows_v[j, r, pl.ds(W - LANES, LANES)] = (
                    1.0 / (1.0 + jnp.exp(-v2)))
            return 0

        lax.fori_loop(0, CH, plane_body, 0)

        pltpu.sync_copy(rows_v, out_hbm.at[pl.ds(base + chunk * CH, CH)])


@jax.jit
def _gather_sigmoid(x_planes, labels_padded):
    mesh = plsc.VectorSubcoreMesh(core_axis_name="c", subcore_axis_name="s")
    return pl.kernel(
        _sc_body,
        out_type=jax.ShapeDtypeStruct((NPAD, H, W), jnp.float32),
        mesh=mesh,
        scratch_types=[
            pltpu.VMEM((BPW,), jnp.int32),
            pltpu.VMEM((CH, H, W), jnp.float32),
            pltpu.SemaphoreType.DMA,
        ],
    )(x_planes, labels_padded)


def kernel(x, labels):
    lab = jnp.pad(labels.astype(jnp.int32), (0, NPAD - N))
    out = _gather_sigmoid(x, lab)
    return out[:N].reshape(N, 1, H, W)


# speedup vs baseline: 28.9425x; 12.1340x over previous
import jax
import jax.numpy as jnp
from jax import lax
from jax.experimental import pallas as pl
from jax.experimental.pallas import tpu as pltpu

N, C, H, W = 1000, 81, 28, 28
HW = H * W
HB = 28


def _tc_body(lab_ref, xt_ref, o_ref):
    lab = lab_ref[...]
    cio = lax.broadcasted_iota(jnp.int32, (C, N), 0)
    m = cio == lab
    for hb in range(HB):
        slab = xt_ref[hb]
        red = jnp.sum(jnp.where(m, slab, 0.0), axis=0, keepdims=True)
        o_ref[0, pl.ds(hb, 1), :] = 1.0 / (1.0 + jnp.exp(-red))


@jax.jit
def _select_sigmoid(lab2, xt):
    return pl.pallas_call(
        _tc_body,
        grid=(HW // HB,),
        in_specs=[
            pl.BlockSpec((1, N), lambda i: (0, 0)),
            pl.BlockSpec((HB, C, N), lambda i: (i, 0, 0)),
        ],
        out_specs=pl.BlockSpec((1, HB, N), lambda i: (i, 0, 0)),
        out_shape=jax.ShapeDtypeStruct((H, W, N), jnp.float32),
        compiler_params=pltpu.CompilerParams(
            dimension_semantics=("arbitrary",),
        ),
    )(lab2, xt)


def kernel(x, labels):
    xt = x.transpose(2, 3, 1, 0).reshape(HW, C, N)
    lab2 = labels.astype(jnp.int32).reshape(1, N)
    o = _select_sigmoid(lab2, xt)
    return o.transpose(2, 0, 1).reshape(N, 1, H, W)
